# baseline (speedup 1.0000x reference)
import functools

import jax
import jax.numpy as jnp
from jax import lax
from jax.experimental import pallas as pl
from jax.experimental.pallas import tpu as pltpu
from jax.experimental.pallas import tpu_sc as plsc


def kernel(labels, train, table):
    del train
    B = labels.shape[0]
    V, D = table.shape
    info = plsc.get_sparse_core_info()
    NC, NS = info.num_cores, info.num_subcores
    NW = NC * NS
    b_per_w = B // NW

    G = 16
    NCH = b_per_w // G

    mesh = plsc.VectorSubcoreMesh(core_axis_name="c", subcore_axis_name="s")

    @functools.partial(
        pl.kernel,
        mesh=mesh,
        out_type=jax.ShapeDtypeStruct((B, D), jnp.float32),
        scratch_types=[
            pltpu.VMEM((b_per_w,), jnp.int32),
            pltpu.SemaphoreType.DMA,
        ],
    )
    def emb(table_hbm, idx_hbm, out_hbm, idx_s, sem):
        wid = lax.axis_index("s") * NC + lax.axis_index("c")
        base = wid * b_per_w
        pltpu.sync_copy(idx_hbm.at[wid], idx_s)

        @pl.loop(0, NCH)
        def _(ch):
            vec = idx_s[pl.ds(ch * G, G)]
            for s in range(G):
                pltpu.async_copy(
                    table_hbm.at[pl.ds(vec[s], 1)],
                    out_hbm.at[pl.ds(base + ch * G + s, 1)],
                    sem,
                )

        @pl.loop(0, b_per_w)
        def _(i):
            pltpu.make_async_copy(
                table_hbm.at[pl.ds(0, 1)], out_hbm.at[pl.ds(base, 1)], sem
            ).wait()

    return emb(table, labels.reshape(NW, b_per_w))
